# baseline (speedup 1.0000x reference)
import functools

import jax
import jax.numpy as jnp
from jax.experimental import pallas as pl
from jax.experimental.pallas import tpu as pltpu

N_ROWS = 4096
N_COLS = 200
BLOCK_R = 8


def _adrmse_block_kernel(x_ref, out_ref):
    x = x_ref[...]

    m = jnp.max(x, axis=1, keepdims=True)
    e = jnp.exp(x - m)
    p = e / jnp.sum(e, axis=1, keepdims=True)
    row_ids = jax.lax.broadcasted_iota(jnp.int32, (N_COLS, N_COLS), 0)
    col_ids = jax.lax.broadcasted_iota(jnp.int32, (N_COLS, N_COLS), 1)
    upper_tri = (row_ids <= col_ids).astype(jnp.float32)
    ar = jax.lax.dot(p, upper_tri,
                     precision=jax.lax.Precision.HIGHEST)

    a = x[:, :, None]
    b = x[:, None, :]
    i_ids = jax.lax.broadcasted_iota(jnp.int32, (BLOCK_R, N_COLS, N_COLS), 1)
    j_ids = jax.lax.broadcasted_iota(jnp.int32, (BLOCK_R, N_COLS, N_COLS), 2)
    beats = (b > a) | ((b == a) & (j_ids < i_ids))
    rank = 1.0 + jnp.sum(beats.astype(jnp.float32), axis=2)

    d = (rank - ar) ** 2 / jnp.log2(rank + 1.0)
    partial = jnp.sum(d)

    @pl.when(pl.program_id(0) == 0)
    def _():
        out_ref[0, 0] = 0.0

    out_ref[0, 0] += partial


@jax.jit
def kernel(scores):
    total = pl.pallas_call(
        _adrmse_block_kernel,
        grid=(N_ROWS // BLOCK_R,),
        in_specs=[pl.BlockSpec((BLOCK_R, N_COLS), lambda i: (i, 0))],
        out_specs=pl.BlockSpec(memory_space=pltpu.SMEM),
        out_shape=jax.ShapeDtypeStruct((1, 1), jnp.float32),
    )(scores)
    return total[0, 0] / (N_ROWS * N_COLS)


# speedup vs baseline: 1.0532x; 1.0532x over previous
import functools

import jax
import jax.numpy as jnp
from jax.experimental import pallas as pl
from jax.experimental.pallas import tpu as pltpu

N_ROWS = 4096
N_COLS = 200
BLOCK_R = 8


def _adrmse_block_kernel(x_ref, out_ref):
    x = x_ref[...]

    m = jnp.max(x, axis=1, keepdims=True)
    e = jnp.exp(x - m)
    p = e / jnp.sum(e, axis=1, keepdims=True)
    row_ids = jax.lax.broadcasted_iota(jnp.int32, (N_COLS, N_COLS), 0)
    col_ids = jax.lax.broadcasted_iota(jnp.int32, (N_COLS, N_COLS), 1)
    upper_tri = (row_ids <= col_ids).astype(jnp.float32)
    ar = jax.lax.dot(p, upper_tri,
                     precision=jax.lax.Precision.HIGHEST)

    bits = jax.lax.bitcast_convert_type(x, jnp.int32)
    k = bits ^ ((bits >> 31) & jnp.int32(0x7FFFFFFF))
    tri_lt = (col_ids < row_ids).astype(jnp.int32)
    w = k[:, :, None] - tri_lt[None, :, :]
    beats = k[:, None, :] > w
    rank = 1.0 + jnp.sum(beats.astype(jnp.float32), axis=2)

    d = (rank - ar) ** 2 / jnp.log2(rank + 1.0)
    partial = jnp.sum(d)

    @pl.when(pl.program_id(0) == 0)
    def _():
        out_ref[0, 0] = 0.0

    out_ref[0, 0] += partial


@jax.jit
def kernel(scores):
    total = pl.pallas_call(
        _adrmse_block_kernel,
        grid=(N_ROWS // BLOCK_R,),
        in_specs=[pl.BlockSpec((BLOCK_R, N_COLS), lambda i: (i, 0))],
        out_specs=pl.BlockSpec(memory_space=pltpu.SMEM),
        out_shape=jax.ShapeDtypeStruct((1, 1), jnp.float32),
    )(scores)
    return total[0, 0] / (N_ROWS * N_COLS)


# speedup vs baseline: 4.6333x; 4.3993x over previous
import jax
import jax.numpy as jnp
from jax.experimental import pallas as pl
from jax.experimental.pallas import tpu as pltpu

N_ROWS = 4096
N_COLS = 200
BLOCK_C = 128
N_CHUNKS = N_COLS // 8


def _adrmse_kernel(xt_ref, tril_ref, out_ref, kscratch_ref):
    xt = xt_ref[...]

    bits = jax.lax.bitcast_convert_type(xt, jnp.int32)
    kt = bits ^ ((bits >> 31) & jnp.int32(0x7FFFFFFF))
    kscratch_ref[...] = kt
    kt_cs = [kt[8 * c:8 * c + 8, :] for c in range(N_CHUNKS)]

    sub_iota = jax.lax.broadcasted_iota(jnp.int32, (8, BLOCK_C), 0)
    accs = [jnp.zeros((8, BLOCK_C), jnp.int32) for _ in range(N_CHUNKS)]
    for cj in range(N_CHUNKS):
        def body(jj, carry, cj=cj):
            bc = jnp.broadcast_to(
                kscratch_ref[pl.ds(8 * cj + jj, 1), :], (8, BLOCK_C))
            bcp = bc + 1
            w_diag = jnp.where(sub_iota > jj, bcp, bc)
            out = []
            for c in range(N_CHUNKS):
                w = bc if c < cj else (bcp if c > cj else w_diag)
                out.append(carry[c] + (w > kt_cs[c]).astype(jnp.int32))
            return out
        accs = jax.lax.fori_loop(0, 8, body, accs)
    rank = 1.0 + jnp.concatenate(accs, axis=0).astype(jnp.float32)

    m = jnp.max(xt, axis=0, keepdims=True)
    e = jnp.exp(xt - m)
    p = e / jnp.sum(e, axis=0, keepdims=True)
    ar = jax.lax.dot(tril_ref[...], p, precision=jax.lax.Precision.HIGHEST)

    d = (rank - ar) ** 2 / jnp.log2(rank + 1.0)
    partial = jnp.sum(d)

    @pl.when(pl.program_id(0) == 0)
    def _():
        out_ref[0, 0] = 0.0

    out_ref[0, 0] += partial


@jax.jit
def kernel(scores):
    xt = scores.T
    ii = jax.lax.broadcasted_iota(jnp.int32, (N_COLS, N_COLS), 0)
    jj = jax.lax.broadcasted_iota(jnp.int32, (N_COLS, N_COLS), 1)
    tril = (ii >= jj).astype(jnp.float32)
    total = pl.pallas_call(
        _adrmse_kernel,
        grid=(N_ROWS // BLOCK_C,),
        in_specs=[
            pl.BlockSpec((N_COLS, BLOCK_C), lambda i: (0, i)),
            pl.BlockSpec((N_COLS, N_COLS), lambda i: (0, 0)),
        ],
        out_specs=pl.BlockSpec(memory_space=pltpu.SMEM),
        out_shape=jax.ShapeDtypeStruct((1, 1), jnp.float32),
        scratch_shapes=[pltpu.VMEM((N_COLS, BLOCK_C), jnp.int32)],
    )(xt, tril)
    return total[0, 0] / (N_ROWS * N_COLS)


# speedup vs baseline: 6.6446x; 1.4341x over previous
import jax
import jax.numpy as jnp
from jax.experimental import pallas as pl
from jax.experimental.pallas import tpu as pltpu

N_ROWS = 4096
N_COLS = 200
BLOCK_C = 128
N_CHUNKS = N_COLS // 8


def _adrmse_kernel(xt_ref, tril_ref, out_ref, kscratch_ref):
    xt = xt_ref[...]

    bits = jax.lax.bitcast_convert_type(xt, jnp.int32)
    kt = bits ^ ((bits >> 31) & jnp.int32(0x7FFFFFFF))
    kscratch_ref[...] = kt
    kt_cs = [kt[8 * c:8 * c + 8, :] for c in range(N_CHUNKS)]

    sub_iota = jax.lax.broadcasted_iota(jnp.int32, (8, BLOCK_C), 0)
    accs = [jnp.zeros((8, BLOCK_C), jnp.int32) for _ in range(N_CHUNKS)]
    for cj in range(N_CHUNKS):
        def body(jj, carry, cj=cj):
            bc = jnp.broadcast_to(
                kscratch_ref[pl.ds(8 * cj + jj, 1), :], (8, BLOCK_C))
            bcp = bc + 1
            w_diag = jnp.where(sub_iota > jj, bcp, bc)
            out = []
            for c in range(N_CHUNKS):
                w = bc if c < cj else (bcp if c > cj else w_diag)
                out.append(carry[c] + (w > kt_cs[c]).astype(jnp.int32))
            return out
        accs = jax.lax.fori_loop(0, 8, body, accs, unroll=4)
    rank = 1.0 + jnp.concatenate(accs, axis=0).astype(jnp.float32)

    m = jnp.max(xt, axis=0, keepdims=True)
    e = jnp.exp(xt - m)
    p = e / jnp.sum(e, axis=0, keepdims=True)
    ar = jax.lax.dot(tril_ref[...], p)

    d = (rank - ar) ** 2 / jnp.log2(rank + 1.0)
    partial = jnp.sum(d)

    @pl.when(pl.program_id(0) == 0)
    def _():
        out_ref[0, 0] = 0.0

    out_ref[0, 0] += partial


@jax.jit
def kernel(scores):
    xt = scores.T
    ii = jax.lax.broadcasted_iota(jnp.int32, (N_COLS, N_COLS), 0)
    jj = jax.lax.broadcasted_iota(jnp.int32, (N_COLS, N_COLS), 1)
    tril = (ii >= jj).astype(jnp.float32)
    total = pl.pallas_call(
        _adrmse_kernel,
        grid=(N_ROWS // BLOCK_C,),
        in_specs=[
            pl.BlockSpec((N_COLS, BLOCK_C), lambda i: (0, i)),
            pl.BlockSpec((N_COLS, N_COLS), lambda i: (0, 0)),
        ],
        out_specs=pl.BlockSpec(memory_space=pltpu.SMEM),
        out_shape=jax.ShapeDtypeStruct((1, 1), jnp.float32),
        scratch_shapes=[pltpu.VMEM((N_COLS, BLOCK_C), jnp.int32)],
    )(xt, tril)
    return total[0, 0] / (N_ROWS * N_COLS)


# speedup vs baseline: 8.1011x; 1.2192x over previous
import jax
import jax.numpy as jnp
from jax.experimental import pallas as pl
from jax.experimental.pallas import tpu as pltpu

N_ROWS = 4096
N_COLS = 200
BLOCK_C = 128
N_CHUNKS = N_COLS // 8


def _adrmse_kernel(xt_ref, tril_ref, out_ref, kscratch_ref):
    xt = xt_ref[...]

    bits = jax.lax.bitcast_convert_type(xt, jnp.int32)
    kt = bits ^ ((bits >> 31) & jnp.int32(0x7FFFFFFF))
    kscratch_ref[...] = kt
    kt_cs = [kt[8 * c:8 * c + 8, :] for c in range(N_CHUNKS)]

    sub_iota = jax.lax.broadcasted_iota(jnp.int32, (8, BLOCK_C), 0)
    accs = [jnp.zeros((8, BLOCK_C), jnp.int32) for _ in range(N_CHUNKS)]
    for cj in range(N_CHUNKS):
        def body(jj, carry, cj=cj):
            bc = jnp.broadcast_to(
                kscratch_ref[pl.ds(8 * cj + jj, 1), :], (8, BLOCK_C))
            bcp = bc + 1
            w_diag = jnp.where(sub_iota > jj, bcp, bc)
            out = []
            for c in range(N_CHUNKS):
                w = bc if c < cj else (bcp if c > cj else w_diag)
                out.append(carry[c] + (w > kt_cs[c]).astype(jnp.int32))
            return out
        accs = jax.lax.fori_loop(0, 8, body, accs, unroll=8)
    rank = 1.0 + jnp.concatenate(accs, axis=0).astype(jnp.float32)

    m = jnp.max(xt, axis=0, keepdims=True)
    e = jnp.exp(xt - m)
    p = e / jnp.sum(e, axis=0, keepdims=True)
    ar = jax.lax.dot(tril_ref[...], p)

    d = (rank - ar) ** 2 / jnp.log2(rank + 1.0)
    partial = jnp.sum(d)

    @pl.when(pl.program_id(0) == 0)
    def _():
        out_ref[0, 0] = 0.0

    out_ref[0, 0] += partial


@jax.jit
def kernel(scores):
    xt = scores.T
    ii = jax.lax.broadcasted_iota(jnp.int32, (N_COLS, N_COLS), 0)
    jj = jax.lax.broadcasted_iota(jnp.int32, (N_COLS, N_COLS), 1)
    tril = (ii >= jj).astype(jnp.float32)
    total = pl.pallas_call(
        _adrmse_kernel,
        grid=(N_ROWS // BLOCK_C,),
        in_specs=[
            pl.BlockSpec((N_COLS, BLOCK_C), lambda i: (0, i)),
            pl.BlockSpec((N_COLS, N_COLS), lambda i: (0, 0)),
        ],
        out_specs=pl.BlockSpec(memory_space=pltpu.SMEM),
        out_shape=jax.ShapeDtypeStruct((1, 1), jnp.float32),
        scratch_shapes=[pltpu.VMEM((N_COLS, BLOCK_C), jnp.int32)],
    )(xt, tril)
    return total[0, 0] / (N_ROWS * N_COLS)
